# baseline (speedup 1.0000x reference)
import functools

import jax
import jax.numpy as jnp
import numpy as np
from jax import lax
from jax.experimental import pallas as pl
from jax.experimental.pallas import tpu as pltpu
from jax.experimental.pallas import tpu_sc as plsc

N_NODES = 10000
D_FEAT = 128
N_EDGES = 320000
NUM_HOPS = 3
SIGMA = 0.1

NC = 2
NS = 16
NW = NC * NS
CHUNK = 128
NPHASE = 2
NCHUNK_P = 40
NCHUNK = NPHASE * NCHUNK_P
EDGES_PAD = NW * NCHUNK * CHUNK
PAD = EDGES_PAD - N_EDGES
ACC_ROWS = 10112
STRIPE = ACC_ROWS // NS

_ROW_BLK = 2000


def _sc_hop_body(h_hbm, src_hbm, dst_hbm, zero_hbm, out_hbm,
                 src_v, dst_v, buf0, buf1, acc, sem0, sem1, ssem0, ssem1):
    cid = lax.axis_index("c")
    sid = lax.axis_index("s")
    wid = sid * NC + cid

    pltpu.sync_copy(zero_hbm, buf0)
    base = sid * STRIPE
    for k in range(STRIPE // CHUNK):
        pltpu.sync_copy(buf0, acc.at[pl.ds(base + k * CHUNK, CHUNK)])
    rem = STRIPE % CHUNK
    if rem:
        pltpu.sync_copy(buf0.at[pl.ds(0, rem)],
                        acc.at[pl.ds(base + (STRIPE // CHUNK) * CHUNK, rem)])
    plsc.subcore_barrier()

    for p in range(NPHASE):
        pltpu.sync_copy(src_hbm.at[wid, p], src_v)
        pltpu.sync_copy(dst_hbm.at[wid, p], dst_v)

        pltpu.async_copy(h_hbm.at[src_v.at[0]], buf0, sem0)
        pltpu.async_copy(h_hbm.at[src_v.at[1]], buf1, sem1)

        def pair(i, carry):
            j0 = 2 * i
            j1 = j0 + 1
            pltpu.make_async_copy(h_hbm.at[src_v.at[j0]], buf0, sem0).wait()
            pltpu.async_copy(buf0, acc.at[dst_v.at[j0]], ssem0, add=True)

            pltpu.make_async_copy(h_hbm.at[src_v.at[j1]], buf1, sem1).wait()
            pltpu.async_copy(buf1, acc.at[dst_v.at[j1]], ssem1, add=True)

            @pl.when(j0 + 2 < NCHUNK_P)
            def _():
                pltpu.make_async_copy(buf0, acc.at[dst_v.at[j0]], ssem0).wait()
                pltpu.async_copy(h_hbm.at[src_v.at[j0 + 2]], buf0, sem0)

            @pl.when(j1 + 2 < NCHUNK_P)
            def _():
                pltpu.make_async_copy(buf1, acc.at[dst_v.at[j1]], ssem1).wait()
                pltpu.async_copy(h_hbm.at[src_v.at[j1 + 2]], buf1, sem1)

            return carry

        lax.fori_loop(0, NCHUNK_P // 2, pair, 0)
        pltpu.make_async_copy(buf0, acc.at[dst_v.at[NCHUNK_P - 2]],
                              ssem0).wait()
        pltpu.make_async_copy(buf1, acc.at[dst_v.at[NCHUNK_P - 1]],
                              ssem1).wait()
    plsc.subcore_barrier()

    out_base = cid * ACC_ROWS + base
    pltpu.sync_copy(acc.at[pl.ds(base, STRIPE)],
                    out_hbm.at[pl.ds(out_base, STRIPE)])


@functools.lru_cache(maxsize=None)
def _make_sc_hop(interpret: bool = False):
    mesh = plsc.VectorSubcoreMesh(core_axis_name="c", subcore_axis_name="s",
                                  num_cores=NC, num_subcores=NS)
    return functools.partial(
        pl.kernel,
        out_type=jax.ShapeDtypeStruct((NC * ACC_ROWS, D_FEAT), jnp.float32),
        mesh=mesh,
        scratch_types=[
            pltpu.VMEM((NCHUNK_P, CHUNK), jnp.int32),
            pltpu.VMEM((NCHUNK_P, CHUNK), jnp.int32),
            pltpu.VMEM((CHUNK, D_FEAT), jnp.float32),
            pltpu.VMEM((CHUNK, D_FEAT), jnp.float32),
            pltpu.VMEM_SHARED((ACC_ROWS, D_FEAT), jnp.float32),
            pltpu.SemaphoreType.DMA,
            pltpu.SemaphoreType.DMA,
            pltpu.SemaphoreType.DMA,
            pltpu.SemaphoreType.DMA,
        ],
        interpret=interpret,
    )(_sc_hop_body)


def _norm_body(x_ref, o_ref):
    t = x_ref[...]
    ss = jnp.sum(t * t, axis=1, keepdims=True)
    o_ref[...] = t / jnp.maximum(jnp.sqrt(ss), 1e-12)


def _finish_body(p_ref, nz_ref, o_ref):
    t = p_ref[0] + p_ref[1] + nz_ref[...]
    ss = jnp.sum(t * t, axis=1, keepdims=True)
    o_ref[...] = t / jnp.maximum(jnp.sqrt(ss), 1e-12)


@functools.lru_cache(maxsize=None)
def _make_tc_kernels(interpret: bool = False):
    grid = (N_NODES // _ROW_BLK,)
    norm = pl.pallas_call(
        _norm_body,
        grid=grid,
        in_specs=[pl.BlockSpec((_ROW_BLK, D_FEAT), lambda i: (i, 0))],
        out_specs=pl.BlockSpec((_ROW_BLK, D_FEAT), lambda i: (i, 0)),
        out_shape=jax.ShapeDtypeStruct((N_NODES, D_FEAT), jnp.float32),
        interpret=interpret,
    )
    finish = pl.pallas_call(
        _finish_body,
        grid=grid,
        in_specs=[
            pl.BlockSpec((NC, _ROW_BLK, D_FEAT), lambda i: (0, i, 0)),
            pl.BlockSpec((_ROW_BLK, D_FEAT), lambda i: (i, 0)),
        ],
        out_specs=pl.BlockSpec((_ROW_BLK, D_FEAT), lambda i: (i, 0)),
        out_shape=jax.ShapeDtypeStruct((N_NODES, D_FEAT), jnp.float32),
        interpret=interpret,
    )
    return norm, finish


def _noise_const():
    key = jax.random.key(42)
    ns = []
    for _ in range(NUM_HOPS):
        key, sub = jax.random.split(key)
        ns.append(SIGMA * jax.random.normal(sub, (N_NODES, D_FEAT),
                                            dtype=jnp.float32))
    return jnp.stack(ns)


def kernel(x, edge_index):
    src = edge_index[0].astype(jnp.int32)
    dst = edge_index[1].astype(jnp.int32)
    pad_i = jnp.arange(PAD, dtype=jnp.int32)
    src_t = jnp.concatenate([src, pad_i % N_NODES]).reshape(
        NW, NPHASE, NCHUNK_P, CHUNK)
    dst_t = jnp.concatenate(
        [dst, N_NODES + pad_i % (ACC_ROWS - N_NODES)]
    ).reshape(NW, NPHASE, NCHUNK_P, CHUNK)
    zeros = jnp.zeros((CHUNK, D_FEAT), jnp.float32)
    noise = _noise_const()

    sc_hop = _make_sc_hop()
    norm, finish = _make_tc_kernels()

    h = norm(x)
    outs = [h]
    for k in range(NUM_HOPS):
        parts = sc_hop(h, src_t, dst_t, zeros)
        h = finish(parts.reshape(NC, ACC_ROWS, D_FEAT), noise[k])
        outs.append(h)
    return jnp.stack(outs)


# speedup vs baseline: 1.2403x; 1.2403x over previous
import functools

import jax
import jax.numpy as jnp
import numpy as np
from jax import lax
from jax.experimental import pallas as pl
from jax.experimental.pallas import tpu as pltpu
from jax.experimental.pallas import tpu_sc as plsc

N_NODES = 10000
D_FEAT = 128
N_EDGES = 320000
NUM_HOPS = 3
SIGMA = 0.1

NC = 2
NS = 16
NW = NC * NS
CHUNK = 128
NPHASE = 2
NCHUNK_P = 40
NCHUNK = NPHASE * NCHUNK_P
EDGES_PAD = NW * NCHUNK * CHUNK
PAD = EDGES_PAD - N_EDGES
ACC_ROWS = 10112
STRIPE = ACC_ROWS // NS

_ROW_BLK = 2000


def _sc_hop_body(h_hbm, src_hbm, dst_hbm, zero_hbm, out_hbm,
                 src_v, dst_v, buf0, buf1, acc, sem0, sem1):
    cid = lax.axis_index("c")
    sid = lax.axis_index("s")
    wid = sid * NC + cid

    pltpu.sync_copy(zero_hbm, buf0)
    base = sid * STRIPE
    for k in range(STRIPE // CHUNK):
        pltpu.sync_copy(buf0, acc.at[pl.ds(base + k * CHUNK, CHUNK)])
    rem = STRIPE % CHUNK
    if rem:
        pltpu.sync_copy(buf0.at[pl.ds(0, rem)],
                        acc.at[pl.ds(base + (STRIPE // CHUNK) * CHUNK, rem)])
    plsc.subcore_barrier()

    for p in range(NPHASE):
        pltpu.sync_copy(src_hbm.at[wid, p], src_v)
        pltpu.sync_copy(dst_hbm.at[wid, p], dst_v)

        pltpu.async_copy(h_hbm.at[src_v.at[0]], buf0, sem0)
        pltpu.async_copy(h_hbm.at[src_v.at[1]], buf1, sem1)

        def pair(i, carry):
            j0 = 2 * i
            j1 = j0 + 1
            pltpu.make_async_copy(h_hbm.at[src_v.at[j0]], buf0, sem0).wait()
            pltpu.sync_copy(buf0, acc.at[dst_v.at[j0]], add=True)
            pltpu.async_copy(h_hbm.at[src_v.at[j0 + 2]], buf0, sem0)

            pltpu.make_async_copy(h_hbm.at[src_v.at[j1]], buf1, sem1).wait()
            pltpu.sync_copy(buf1, acc.at[dst_v.at[j1]], add=True)
            pltpu.async_copy(h_hbm.at[src_v.at[j1 + 2]], buf1, sem1)

            return carry

        lax.fori_loop(0, NCHUNK_P // 2 - 1, pair, 0)
        jl0 = NCHUNK_P - 2
        jl1 = NCHUNK_P - 1
        pltpu.make_async_copy(h_hbm.at[src_v.at[jl0]], buf0, sem0).wait()
        pltpu.sync_copy(buf0, acc.at[dst_v.at[jl0]], add=True)
        pltpu.make_async_copy(h_hbm.at[src_v.at[jl1]], buf1, sem1).wait()
        pltpu.sync_copy(buf1, acc.at[dst_v.at[jl1]], add=True)
    plsc.subcore_barrier()

    out_base = cid * ACC_ROWS + base
    pltpu.sync_copy(acc.at[pl.ds(base, STRIPE)],
                    out_hbm.at[pl.ds(out_base, STRIPE)])


@functools.lru_cache(maxsize=None)
def _make_sc_hop(interpret: bool = False):
    mesh = plsc.VectorSubcoreMesh(core_axis_name="c", subcore_axis_name="s",
                                  num_cores=NC, num_subcores=NS)
    return functools.partial(
        pl.kernel,
        out_type=jax.ShapeDtypeStruct((NC * ACC_ROWS, D_FEAT), jnp.float32),
        mesh=mesh,
        scratch_types=[
            pltpu.VMEM((NCHUNK_P, CHUNK), jnp.int32),
            pltpu.VMEM((NCHUNK_P, CHUNK), jnp.int32),
            pltpu.VMEM((CHUNK, D_FEAT), jnp.float32),
            pltpu.VMEM((CHUNK, D_FEAT), jnp.float32),
            pltpu.VMEM_SHARED((ACC_ROWS, D_FEAT), jnp.float32),
            pltpu.SemaphoreType.DMA,
            pltpu.SemaphoreType.DMA,
        ],
        interpret=interpret,
    )(_sc_hop_body)


def _norm_body(x_ref, o_ref):
    t = x_ref[...]
    ss = jnp.sum(t * t, axis=1, keepdims=True)
    o_ref[...] = t / jnp.maximum(jnp.sqrt(ss), 1e-12)


def _finish_body(p_ref, nz_ref, o_ref):
    t = p_ref[0] + p_ref[1] + nz_ref[...]
    ss = jnp.sum(t * t, axis=1, keepdims=True)
    o_ref[...] = t / jnp.maximum(jnp.sqrt(ss), 1e-12)


@functools.lru_cache(maxsize=None)
def _make_tc_kernels(interpret: bool = False):
    grid = (N_NODES // _ROW_BLK,)
    norm = pl.pallas_call(
        _norm_body,
        grid=grid,
        in_specs=[pl.BlockSpec((_ROW_BLK, D_FEAT), lambda i: (i, 0))],
        out_specs=pl.BlockSpec((_ROW_BLK, D_FEAT), lambda i: (i, 0)),
        out_shape=jax.ShapeDtypeStruct((N_NODES, D_FEAT), jnp.float32),
        interpret=interpret,
    )
    finish = pl.pallas_call(
        _finish_body,
        grid=grid,
        in_specs=[
            pl.BlockSpec((NC, _ROW_BLK, D_FEAT), lambda i: (0, i, 0)),
            pl.BlockSpec((_ROW_BLK, D_FEAT), lambda i: (i, 0)),
        ],
        out_specs=pl.BlockSpec((_ROW_BLK, D_FEAT), lambda i: (i, 0)),
        out_shape=jax.ShapeDtypeStruct((N_NODES, D_FEAT), jnp.float32),
        interpret=interpret,
    )
    return norm, finish


def _noise_const():
    key = jax.random.key(42)
    ns = []
    for _ in range(NUM_HOPS):
        key, sub = jax.random.split(key)
        ns.append(SIGMA * jax.random.normal(sub, (N_NODES, D_FEAT),
                                            dtype=jnp.float32))
    return jnp.stack(ns)


def kernel(x, edge_index):
    src = edge_index[0].astype(jnp.int32)
    dst = edge_index[1].astype(jnp.int32)
    pad_i = jnp.arange(PAD, dtype=jnp.int32)
    src_t = jnp.concatenate([src, pad_i % N_NODES]).reshape(
        NW, NPHASE, NCHUNK_P, CHUNK)
    dst_t = jnp.concatenate(
        [dst, N_NODES + pad_i % (ACC_ROWS - N_NODES)]
    ).reshape(NW, NPHASE, NCHUNK_P, CHUNK)
    zeros = jnp.zeros((CHUNK, D_FEAT), jnp.float32)
    noise = _noise_const()

    sc_hop = _make_sc_hop()
    norm, finish = _make_tc_kernels()

    h = norm(x)
    outs = [h]
    for k in range(NUM_HOPS):
        parts = sc_hop(h, src_t, dst_t, zeros)
        h = finish(parts.reshape(NC, ACC_ROWS, D_FEAT), noise[k])
        outs.append(h)
    return jnp.stack(outs)
